# baseline (speedup 1.0000x reference)
import jax
import jax.numpy as jnp
from jax import lax
from jax.experimental import pallas as pl
from jax.experimental.pallas import tpu as pltpu
from jax.experimental.pallas import tpu_sc as plsc

NC = 2
NS = 16
NW = NC * NS
LANES = 16
CHUNK = 64
NBUF = 5


def _sc_body(pos_hbm, neg_hbm, emb_hbm, out_hbm, pair0_v, pair1_v, *rest):
  row_bufs = rest[:NBUF]
  acc_v = rest[NBUF]
  sems = rest[NBUF + 1:]
  n_chunks = pair0_v.shape[0]

  wid = lax.axis_index("s") * NC + lax.axis_index("c")

  pltpu.sync_copy(pos_hbm.at[wid], pair0_v)
  pltpu.sync_copy(neg_hbm.at[wid], pair1_v)

  pairs = [pair0_v, pair1_v]
  steps = [(g, c) for g in range(2) for c in range(n_chunks)]

  def start(s):
    g, c = steps[s]
    b = s % NBUF
    return pltpu.async_copy(
        emb_hbm.at[pairs[g].at[c]], row_bufs[b], sems[b])

  inflight = {s: start(s) for s in range(min(NBUF - 1, len(steps)))}

  zero = jnp.zeros((LANES,), jnp.float32)
  for g in range(2):
    accs = tuple(zero for _ in range(24))
    for c in range(n_chunks):
      s = g * n_chunks + c
      nxt = s + NBUF - 1
      if nxt < len(steps):
        inflight[nxt] = start(nxt)
      inflight.pop(s).wait()
      rb = row_bufs[s % NBUF]

      def body(i2, carry, rb=rb):
        a = list(carry)
        for u in range(2):
          i = 2 * (2 * i2 + u)
          for j in range(8):
            o = rb[i, pl.ds(j * LANES, LANES)]
            d = rb[i + 1, pl.ds(j * LANES, LANES)]
            a[3 * j + 0] = a[3 * j + 0] + o * d
            a[3 * j + 1] = a[3 * j + 1] + o * o
            a[3 * j + 2] = a[3 * j + 2] + d * d
        return tuple(a)

      accs = lax.fori_loop(0, CHUNK // 2, body, accs)

    for j in range(8):
      for k in range(3):
        acc_v[3 * g + k, pl.ds(j * LANES, LANES)] = accs[3 * j + k]

  pltpu.sync_copy(acc_v, out_hbm.at[wid])


def kernel(pos_pair, neg_pair, emb):
  B = pos_pair.shape[0]
  per_w = B // NW
  n_chunks = per_w // CHUNK

  pos3 = pos_pair.astype(jnp.int32).reshape(NW, n_chunks, 2 * CHUNK)
  neg3 = neg_pair.astype(jnp.int32).reshape(NW, n_chunks, 2 * CHUNK)

  mesh = plsc.VectorSubcoreMesh(core_axis_name="c", subcore_axis_name="s",
                                num_cores=NC, num_subcores=NS)
  scratch = [
      pltpu.VMEM((n_chunks, 2 * CHUNK), jnp.int32),
      pltpu.VMEM((n_chunks, 2 * CHUNK), jnp.int32),
  ]
  scratch += [pltpu.VMEM((2 * CHUNK, 128), jnp.float32) for _ in range(NBUF)]
  scratch += [pltpu.VMEM((6, 128), jnp.float32)]
  scratch += [pltpu.SemaphoreType.DMA for _ in range(NBUF)]
  partials = pl.kernel(
      _sc_body,
      out_type=jax.ShapeDtypeStruct((NW, 6, 128), jnp.float32),
      mesh=mesh,
      scratch_types=scratch,
  )(pos3, neg3, emb)

  p = jnp.sum(partials, axis=0)
  eps = jnp.float32(1e-8)

  def cos(num, so, sd):
    return num / (jnp.maximum(jnp.sqrt(so), eps) * jnp.maximum(jnp.sqrt(sd), eps))

  pos_dist = cos(p[0], p[1], p[2])
  neg_dist = cos(p[3], p[4], p[5])
  return (pos_dist, neg_dist)


# speedup vs baseline: 1.5341x; 1.5341x over previous
import jax
import jax.numpy as jnp
from jax import lax
from jax.experimental import pallas as pl
from jax.experimental.pallas import tpu as pltpu
from jax.experimental.pallas import tpu_sc as plsc

NC = 2
NS = 16
NW = NC * NS
LANES = 16
CHUNK = 64
NBUF = 6


def _sc_body(idx_hbm, emb_hbm, out_hbm, idx_v, *rest):
  row_bufs = [(rest[2 * b], rest[2 * b + 1]) for b in range(NBUF)]
  acc_v = rest[2 * NBUF]
  sidx_v = rest[2 * NBUF + 1]
  shared = rest[2 * NBUF + 2]
  sems = rest[2 * NBUF + 3:]
  n_chunks = idx_hbm.shape[2]

  sid = lax.axis_index("s")
  cid = lax.axis_index("c")
  wid = sid * NC + cid

  pltpu.sync_copy(idx_hbm.at[wid], idx_v)

  zero = jnp.zeros((LANES,), jnp.float32)
  for r in range(16):
    for j in range(8):
      acc_v[r, pl.ds(j * LANES, LANES)] = zero
  sidx_v[...] = lax.iota(jnp.int32, LANES)

  @pl.when(sid == 0)
  def _():
    pltpu.sync_copy(acc_v, shared)

  steps = [(g, c) for g in range(2) for c in range(n_chunks)]

  def start(s):
    g, c = steps[s]
    b = s % NBUF
    ro, rd = row_bufs[b]
    ho = pltpu.async_copy(emb_hbm.at[idx_v.at[2 * g, c]], ro, sems[b])
    hd = pltpu.async_copy(emb_hbm.at[idx_v.at[2 * g + 1, c]], rd, sems[b])
    return (ho, hd)

  inflight = {s: start(s) for s in range(min(NBUF - 1, len(steps)))}

  zero = jnp.zeros((LANES,), jnp.float32)
  for g in range(2):
    accs = tuple(zero for _ in range(24))
    for c in range(n_chunks):
      s = g * n_chunks + c
      nxt = s + NBUF - 1
      if nxt < len(steps):
        inflight[nxt] = start(nxt)
      ho, hd = inflight.pop(s)
      ho.wait()
      hd.wait()
      ro, rd = row_bufs[s % NBUF]

      def body(i2, carry, ro=ro, rd=rd):
        a = list(carry)
        for u in range(2):
          i = 2 * i2 + u
          for j in range(8):
            o = ro[i, pl.ds(j * LANES, LANES)]
            d = rd[i, pl.ds(j * LANES, LANES)]
            a[3 * j + 0] = a[3 * j + 0] + o * d
            a[3 * j + 1] = a[3 * j + 1] + o * o
            a[3 * j + 2] = a[3 * j + 2] + d * d
        return tuple(a)

      accs = lax.fori_loop(0, CHUNK // 2, body, accs)

    for j in range(8):
      for k in range(3):
        acc_v[3 * g + k, pl.ds(j * LANES, LANES)] = accs[3 * j + k]

  plsc.subcore_barrier()
  pltpu.sync_copy(acc_v, shared.at[sidx_v], add=True)
  plsc.subcore_barrier()

  @pl.when(sid == 0)
  def _():
    pltpu.sync_copy(shared, acc_v)
    pltpu.sync_copy(acc_v, out_hbm.at[cid])


def kernel(pos_pair, neg_pair, emb):
  B = pos_pair.shape[0]
  per_w = B // NW
  n_chunks = per_w // CHUNK
  idx = jnp.stack([pos_pair[:, 0], pos_pair[:, 1],
                   neg_pair[:, 0], neg_pair[:, 1]]).astype(jnp.int32)
  idx = idx.reshape(4, NW, n_chunks, CHUNK).transpose(1, 0, 2, 3)

  mesh = plsc.VectorSubcoreMesh(core_axis_name="c", subcore_axis_name="s",
                                num_cores=NC, num_subcores=NS)
  scratch = [pltpu.VMEM((4, n_chunks, CHUNK), jnp.int32)]
  scratch += [pltpu.VMEM((CHUNK, 128), jnp.float32) for _ in range(2 * NBUF)]
  scratch += [
      pltpu.VMEM((16, 128), jnp.float32),
      pltpu.VMEM((LANES,), jnp.int32),
      pltpu.VMEM_SHARED((16, 128), jnp.float32),
  ]
  scratch += [pltpu.SemaphoreType.DMA for _ in range(NBUF)]
  partials = pl.kernel(
      _sc_body,
      out_type=jax.ShapeDtypeStruct((NC, 16, 128), jnp.float32),
      mesh=mesh,
      scratch_types=scratch,
  )(idx, emb)

  p = jnp.sum(partials, axis=0)
  eps = jnp.float32(1e-8)

  def cos(num, so, sd):
    return num / (jnp.maximum(jnp.sqrt(so), eps) * jnp.maximum(jnp.sqrt(sd), eps))

  pos_dist = cos(p[0], p[1], p[2])
  neg_dist = cos(p[3], p[4], p[5])
  return (pos_dist, neg_dist)


# speedup vs baseline: 1.5609x; 1.0174x over previous
import jax
import jax.numpy as jnp
from jax import lax
from jax.experimental import pallas as pl
from jax.experimental.pallas import tpu as pltpu
from jax.experimental.pallas import tpu_sc as plsc

NC = 2
NS = 16
NW = NC * NS
LANES = 16
CHUNK = 64
NBUF = 6


def _sc_body(idx_hbm, emb_hbm, out_hbm, idx_v, *rest):
  row_bufs = [(rest[2 * b], rest[2 * b + 1]) for b in range(NBUF)]
  acc_v = rest[2 * NBUF]
  osem = rest[2 * NBUF + 1]
  sems = rest[2 * NBUF + 2:]
  n_chunks = idx_hbm.shape[2]

  wid = lax.axis_index("s") * NC + lax.axis_index("c")

  pltpu.sync_copy(idx_hbm.at[wid], idx_v)

  steps = [(g, c) for g in range(2) for c in range(n_chunks)]

  def start(s):
    g, c = steps[s]
    b = s % NBUF
    ro, rd = row_bufs[b]
    ho = pltpu.async_copy(emb_hbm.at[idx_v.at[2 * g, c]], ro, sems[b])
    hd = pltpu.async_copy(emb_hbm.at[idx_v.at[2 * g + 1, c]], rd, sems[b])
    return (ho, hd)

  inflight = {s: start(s) for s in range(min(NBUF - 1, len(steps)))}
  out_handles = []

  zero = jnp.zeros((LANES,), jnp.float32)
  for g in range(2):
    accs = tuple(zero for _ in range(24))
    for c in range(n_chunks):
      s = g * n_chunks + c
      nxt = s + NBUF - 1
      if nxt < len(steps):
        inflight[nxt] = start(nxt)
      ho, hd = inflight.pop(s)
      ho.wait()
      hd.wait()
      ro, rd = row_bufs[s % NBUF]

      def body(i2, carry, ro=ro, rd=rd):
        a = list(carry)
        for u in range(2):
          i = 2 * i2 + u
          for j in range(8):
            o = ro[i, pl.ds(j * LANES, LANES)]
            d = rd[i, pl.ds(j * LANES, LANES)]
            a[3 * j + 0] = a[3 * j + 0] + o * d
            a[3 * j + 1] = a[3 * j + 1] + o * o
            a[3 * j + 2] = a[3 * j + 2] + d * d
        return tuple(a)

      accs = lax.fori_loop(0, CHUNK // 2, body, accs)

    for j in range(8):
      for k in range(3):
        acc_v[3 * g + k, pl.ds(j * LANES, LANES)] = accs[3 * j + k]

    out_handles.append(pltpu.async_copy(
        acc_v.at[pl.ds(3 * g, 3)], out_hbm.at[wid, pl.ds(3 * g, 3)], osem))

  for h in out_handles:
    h.wait()


def kernel(pos_pair, neg_pair, emb):
  B = pos_pair.shape[0]
  per_w = B // NW
  n_chunks = per_w // CHUNK
  idx = jnp.stack([pos_pair[:, 0], pos_pair[:, 1],
                   neg_pair[:, 0], neg_pair[:, 1]]).astype(jnp.int32)
  idx = idx.reshape(4, NW, n_chunks, CHUNK).transpose(1, 0, 2, 3)

  mesh = plsc.VectorSubcoreMesh(core_axis_name="c", subcore_axis_name="s",
                                num_cores=NC, num_subcores=NS)
  scratch = [pltpu.VMEM((4, n_chunks, CHUNK), jnp.int32)]
  scratch += [pltpu.VMEM((CHUNK, 128), jnp.float32) for _ in range(2 * NBUF)]
  scratch += [pltpu.VMEM((6, 128), jnp.float32)]
  scratch += [pltpu.SemaphoreType.DMA for _ in range(NBUF + 1)]
  partials = pl.kernel(
      _sc_body,
      out_type=jax.ShapeDtypeStruct((NW, 6, 128), jnp.float32),
      mesh=mesh,
      scratch_types=scratch,
  )(idx, emb)

  p = jnp.sum(partials, axis=0)
  eps = jnp.float32(1e-8)

  def cos(num, so, sd):
    return num / (jnp.maximum(jnp.sqrt(so), eps) * jnp.maximum(jnp.sqrt(sd), eps))

  pos_dist = cos(p[0], p[1], p[2])
  neg_dist = cos(p[3], p[4], p[5])
  return (pos_dist, neg_dist)
